# Initial kernel scaffold; baseline (speedup 1.0000x reference)
import jax
import jax.numpy as jnp
from jax import lax
from jax.experimental import pallas as pl
from jax.experimental.pallas import tpu as pltpu
from jax.experimental.pallas import tpu_sc as plsc

N_NODES = 10000
N_EDGES = 160000
H = 256
HH = 128
EPS = 1e-10

RB = 1000
NRB = N_NODES // RB

NSUB = 16
EDGES_PER_TILE = N_EDGES // NSUB
CHUNK = 40
NCHUNK = EDGES_PER_TILE // CHUNK
ROWS_PER_TILE = N_NODES // NSUB


def _tc_encode_body(nodes, xp, wx, b1, W1a, W2, b2, Wt, Wb, bm,
                    h_o, p0_o, p1_o, q0_o, q1_o):
    x = xp[...].reshape(RB, 1)
    t = nodes[...] @ W1a[...] + x * wx[...] + b1[...]
    t = jnp.maximum(t, 0.0)
    h = t @ W2[...] + b2[...]
    h = jnp.maximum(h, 0.0)
    h_o[...] = h
    p = h @ Wt[...]
    q = h @ Wb[...] + bm[...]
    p0_o[...] = p[:, :HH]
    p1_o[...] = p[:, HH:]
    q0_o[...] = q[:, :HH]
    q1_o[...] = q[:, HH:]


def _tc_update_body(h, a0, a1, A, B0, B1, bu, Wt, Wb, bm,
                    h_o, p0_o, p1_o, q0_o, q1_o):
    u = h[...] @ A[...] + a0[...] @ B0[...] + a1[...] @ B1[...] + bu[...]
    hn = h[...] + jnp.maximum(u, 0.0)
    h_o[...] = hn
    p = hn @ Wt[...]
    q = hn @ Wb[...] + bm[...]
    p0_o[...] = p[:, :HH]
    p1_o[...] = p[:, HH:]
    q0_o[...] = q[:, :HH]
    q1_o[...] = q[:, HH:]


def _tc_decode_body(h, a0, a1, A, B0, B1, bu, Wd1, bd1, Wd2, bd2, Wab, bab,
                    ab_o):
    u = h[...] @ A[...] + a0[...] @ B0[...] + a1[...] @ B1[...] + bu[...]
    hn = h[...] + jnp.maximum(u, 0.0)
    d = jnp.maximum(hn @ Wd1[...] + bd1[...], 0.0)
    d = d @ Wd2[...] + bd2[...]
    z = d @ Wab[...] + bab[...]
    ab_o[...] = jnp.where(z > 0, z, jnp.exp(z) - 1.0) + (1.0 + EPS)


def _row_spec(shape):
    return pl.BlockSpec((RB,) + tuple(shape[1:]),
                        lambda i: (i,) + (0,) * (len(shape) - 1))


def _full_spec(shape):
    return pl.BlockSpec(tuple(shape), lambda i: (0,) * len(shape))


def _tc_call(body, ins, out_shapes):
    in_specs = [(_row_spec(x.shape) if x.shape and x.shape[0] == N_NODES
                 else _full_spec(x.shape)) for x in ins]
    out_specs = [_row_spec(s.shape) for s in out_shapes]
    return pl.pallas_call(
        body,
        grid=(NRB,),
        in_specs=in_specs,
        out_specs=out_specs,
        out_shape=tuple(out_shapes),
    )(*ins)


def _sc_edge_run(P, Q, agg_o, s, sidx, ridx, sidx_v, ridx_v,
                 pbuf, qbuf, mbuf, semp, semq, acc):
    pltpu.sync_copy(sidx.at[s], sidx_v)
    pltpu.sync_copy(ridx.at[s], ridx_v)

    def zrow(j, _):
        for f in range(HH // 16):
            mbuf[j, pl.ds(f * 16, 16)] = jnp.zeros((16,), jnp.float32)
        return 0
    lax.fori_loop(0, CHUNK, zrow, 0)
    base = s * ROWS_PER_TILE

    def zcp(k, _):
        pltpu.sync_copy(mbuf, acc.at[pl.ds(base + k * CHUNK, CHUNK)])
        return 0
    lax.fori_loop(0, ROWS_PER_TILE // CHUNK, zcp, 0)
    pltpu.sync_copy(mbuf.at[pl.ds(0, ROWS_PER_TILE % CHUNK)],
                    acc.at[pl.ds(base + 600, ROWS_PER_TILE % CHUNK)])
    plsc.subcore_barrier()

    def chunk(j, _):
        cp = pltpu.async_copy(P.at[sidx_v.at[j]], pbuf, semp)
        cq = pltpu.async_copy(Q.at[ridx_v.at[j]], qbuf, semq)
        cp.wait()
        cq.wait()

        def ebody(e, _):
            for f in range(HH // 16):
                sl = pl.ds(f * 16, 16)
                mbuf[e, sl] = jnp.maximum(pbuf[e, sl] + qbuf[e, sl], 0.0)
            return 0
        lax.fori_loop(0, CHUNK, ebody, 0)
        pltpu.sync_copy(mbuf, acc.at[ridx_v.at[j]], add=True)
        return 0
    lax.fori_loop(0, NCHUNK, chunk, 0)
    plsc.subcore_barrier()

    pltpu.sync_copy(acc.at[pl.ds(base, ROWS_PER_TILE)],
                    agg_o.at[pl.ds(base, ROWS_PER_TILE)])


def _sc_edge_body(P0, P1, Q0, Q1, sidx, ridx, agg0, agg1,
                  sidx_v, ridx_v, pbuf, qbuf, mbuf, semp, semq, acc):
    c = lax.axis_index("c")
    s = lax.axis_index("s")

    @pl.when(c == 0)
    def _():
        _sc_edge_run(P0, Q0, agg0, s, sidx, ridx, sidx_v, ridx_v,
                     pbuf, qbuf, mbuf, semp, semq, acc)

    @pl.when(c == 1)
    def _():
        _sc_edge_run(P1, Q1, agg1, s, sidx, ridx, sidx_v, ridx_v,
                     pbuf, qbuf, mbuf, semp, semq, acc)


_SC_MESH = plsc.VectorSubcoreMesh(core_axis_name="c", subcore_axis_name="s",
                                  num_cores=2, num_subcores=NSUB)

_sc_edge = pl.kernel(
    _sc_edge_body,
    out_type=(jax.ShapeDtypeStruct((N_NODES, HH), jnp.float32),
              jax.ShapeDtypeStruct((N_NODES, HH), jnp.float32)),
    mesh=_SC_MESH,
    scratch_types=[
        pltpu.VMEM((NCHUNK, CHUNK), jnp.int32),
        pltpu.VMEM((NCHUNK, CHUNK), jnp.int32),
        pltpu.VMEM((CHUNK, HH), jnp.float32),
        pltpu.VMEM((CHUNK, HH), jnp.float32),
        pltpu.VMEM((CHUNK, HH), jnp.float32),
        pltpu.SemaphoreType.DMA,
        pltpu.SemaphoreType.DMA,
        pltpu.VMEM_SHARED((N_NODES, HH), jnp.float32),
    ],
)


def kernel(nodes, edge_index, x_prev, t_idx, W_enc1, b_enc1, W_enc2, b_enc2,
           W_msg, b_msg, W_upd, b_upd, W_dec1, b_dec1, W_dec2, b_dec2,
           W_a, b_a, W_b, b_b):
    f32 = jnp.float32
    W1a = W_enc1[:H]
    wx = W_enc1[H].reshape(1, H)
    onehot_row = lax.dynamic_slice_in_dim(W_enc1, H + 1 + t_idx, 1, axis=0)
    b1 = (b_enc1 + onehot_row[0]).reshape(1, H)
    b2 = b_enc2.reshape(1, H)

    sidx = edge_index[0].reshape(NSUB, NCHUNK, CHUNK)
    ridx = edge_index[1].reshape(NSUB, NCHUNK, CHUNK)

    Wab = jnp.zeros((H, HH), f32).at[:, 0].set(W_a[:, 0]).at[:, 1].set(W_b[:, 0])
    bab = jnp.zeros((1, HH), f32).at[0, 0].set(b_a[0]).at[0, 1].set(b_b[0])

    def msg_w(st):
        return (W_msg[st][:H], W_msg[st][H:], b_msg[st].reshape(1, H))

    def upd_w(st):
        return (W_upd[st][:H], W_upd[st][H:H + HH], W_upd[st][H + HH:],
                b_upd[st].reshape(1, H))

    hs = jax.ShapeDtypeStruct((N_NODES, H), f32)
    hh = jax.ShapeDtypeStruct((N_NODES, HH), f32)

    Wt, Wb_, bm = msg_w(0)
    h, P0, P1, Q0, Q1 = _tc_call(
        _tc_encode_body,
        (nodes, x_prev[:, 0], wx, b1, W1a, W_enc2, b2, Wt, Wb_, bm),
        (hs, hh, hh, hh, hh))

    for st in range(3):
        agg0, agg1 = _sc_edge(P0, P1, Q0, Q1, sidx, ridx)
        A, B0, B1, bu = upd_w(st)
        if st < 2:
            Wt, Wb_, bm = msg_w(st + 1)
            h, P0, P1, Q0, Q1 = _tc_call(
                _tc_update_body,
                (h, agg0, agg1, A, B0, B1, bu, Wt, Wb_, bm),
                (hs, hh, hh, hh, hh))
        else:
            (ab,) = _tc_call(
                _tc_decode_body,
                (h, agg0, agg1, A, B0, B1, bu,
                 W_dec1, b_dec1.reshape(1, H), W_dec2, b_dec2.reshape(1, H),
                 Wab, bab),
                (jax.ShapeDtypeStruct((N_NODES, HH), f32),))

    return ab[:, 0], ab[:, 1]


# speedup vs baseline: 1.6343x; 1.6343x over previous
import jax
import jax.numpy as jnp
import functools
from jax import lax
from jax.experimental import pallas as pl
from jax.experimental.pallas import tpu as pltpu
from jax.experimental.pallas import tpu_sc as plsc

N_NODES = 10000
N_EDGES = 160000
H = 256
HH = 128
EPS = 1e-10

RB = 1000
NRB = N_NODES // RB

NSUB = 16
EDGES_PER_TILE = N_EDGES // NSUB
CHUNK = 128
NCHUNK = -(-EDGES_PER_TILE // CHUNK)
PAD_EPT = NCHUNK * CHUNK
HALF_NODES = N_NODES // 2
DUMP_ROW = HALF_NODES
ACC_ROWS = HALF_NODES + 8
ZSTRIPE = 320


def _mm(a, b):
    return jnp.dot(a, b, preferred_element_type=jnp.float32)


def _tc_encode_body(nodes, x2, W1a, W1b, b1, W2, b2, Wt, Wb, bm,
                    h_o, p0, p1, q0, q1):
    t = _mm(nodes[...], W1a[...]) + _mm(x2[...], W1b[...]) + b1[...]
    t = jnp.maximum(t, 0.0)
    h = _mm(t, W2[...]) + b2[...]
    h = jnp.maximum(h, 0.0)
    h_o[...] = h
    p = _mm(h, Wt[...])
    q = _mm(h, Wb[...]) + bm[...]
    p0[...] = p[:, :HH]
    p1[...] = p[:, HH:]
    q0[...] = q[:, :HH]
    q1[...] = q[:, HH:]


def _tc_update_body(h, agg, A, B, bu, Wt, Wb, bm,
                    h_o, p0, p1, q0, q1):
    u = _mm(h[...], A[...]) + _mm(agg[...], B[...]) + bu[...]
    hn = h[...] + jnp.maximum(u, 0.0)
    h_o[...] = hn
    p = _mm(hn, Wt[...])
    q = _mm(hn, Wb[...]) + bm[...]
    p0[...] = p[:, :HH]
    p1[...] = p[:, HH:]
    q0[...] = q[:, :HH]
    q1[...] = q[:, HH:]


def _tc_decode_body(h, agg, A, B, bu,
                    Wd1, bd1, Wd2, bd2, Wab, bab, ab_o):
    u = _mm(h[...], A[...]) + _mm(agg[...], B[...]) + bu[...]
    hn = h[...] + jnp.maximum(u, 0.0)
    d = jnp.maximum(_mm(hn, Wd1[...]) + bd1[...], 0.0)
    d = _mm(d, Wd2[...]) + bd2[...]
    z = _mm(d, Wab[...]) + bab[...]
    ab_o[...] = jnp.where(z > 0, z, jnp.exp(z) - 1.0) + (1.0 + EPS)


def _row_spec(shape):
    return pl.BlockSpec((RB,) + tuple(shape[1:]),
                        lambda i: (i,) + (0,) * (len(shape) - 1))


def _full_spec(shape):
    return pl.BlockSpec(tuple(shape), lambda i: (0,) * len(shape))


def _tc_call(body, ins, out_shapes):
    in_specs = [(_row_spec(x.shape) if x.shape and x.shape[0] == N_NODES
                 else _full_spec(x.shape)) for x in ins]
    out_specs = [_row_spec(s.shape) for s in out_shapes]
    return pl.pallas_call(
        body,
        grid=(NRB,),
        in_specs=in_specs,
        out_specs=out_specs,
        out_shape=tuple(out_shapes),
    )(*ins)


def _sc_edge_run(P, Q, ridxc, agg_o, cbase, s, sidx, ridx,
                 sidx_v, ridx_v, ridxc_v, pbuf, qbuf, mbuf, semp, semq, acc):
    pltpu.sync_copy(sidx.at[s], sidx_v)
    pltpu.sync_copy(ridx.at[s], ridx_v)
    pltpu.sync_copy(ridxc.at[s], ridxc_v)

    def zrow(j, _):
        for f in range(HH // 16):
            mbuf[j, pl.ds(f * 16, 16)] = jnp.zeros((16,), jnp.float32)
        return 0
    lax.fori_loop(0, CHUNK, zrow, 0)
    base = s * ZSTRIPE

    @pl.when(s < NSUB - 1)
    def _():
        pltpu.sync_copy(mbuf, acc.at[pl.ds(base, CHUNK)])
        pltpu.sync_copy(mbuf, acc.at[pl.ds(base + CHUNK, CHUNK)])
        pltpu.sync_copy(mbuf.at[pl.ds(0, ZSTRIPE - 2 * CHUNK)],
                        acc.at[pl.ds(base + 2 * CHUNK, ZSTRIPE - 2 * CHUNK)])

    @pl.when(s == NSUB - 1)
    def _():
        tb = (NSUB - 1) * ZSTRIPE
        pltpu.sync_copy(mbuf, acc.at[pl.ds(tb, CHUNK)])
        pltpu.sync_copy(mbuf.at[pl.ds(0, ACC_ROWS - tb - CHUNK)],
                        acc.at[pl.ds(tb + CHUNK, ACC_ROWS - tb - CHUNK)])
    plsc.subcore_barrier()

    def chunk(j, _):
        cp = pltpu.async_copy(P.at[sidx_v.at[j]], pbuf, semp)
        cq = pltpu.async_copy(Q.at[ridx_v.at[j]], qbuf, semq)
        cp.wait()
        cq.wait()

        def ebody(e, _):
            for f in range(HH // 16):
                sl = pl.ds(f * 16, 16)
                mbuf[e, sl] = jnp.maximum(pbuf[e, sl] + qbuf[e, sl], 0.0)
            return 0
        lax.fori_loop(0, CHUNK, ebody, 0)
        pltpu.sync_copy(mbuf, acc.at[ridxc_v.at[j]], add=True)
        return 0
    lax.fori_loop(0, NCHUNK, chunk, 0)
    plsc.subcore_barrier()

    @pl.when(s < NSUB - 1)
    def _():
        pltpu.sync_copy(acc.at[pl.ds(base, ZSTRIPE)],
                        agg_o.at[pl.ds(cbase + base, ZSTRIPE)])

    @pl.when(s == NSUB - 1)
    def _():
        tb = (NSUB - 1) * ZSTRIPE
        pltpu.sync_copy(acc.at[pl.ds(tb, HALF_NODES - tb)],
                        agg_o.at[pl.ds(cbase + tb, HALF_NODES - tb)])


def _sc_edge_body(P, Q, sidx, ridx, ridxc0, ridxc1, agg_o,
                  sidx_v, ridx_v, ridxc_v, pbuf, qbuf, mbuf,
                  semp, semq, acc):
    c = lax.axis_index("c")
    s = lax.axis_index("s")

    @pl.when(c == 0)
    def _():
        _sc_edge_run(P, Q, ridxc0, agg_o, 0, s, sidx, ridx,
                     sidx_v, ridx_v, ridxc_v, pbuf, qbuf, mbuf,
                     semp, semq, acc)

    @pl.when(c == 1)
    def _():
        _sc_edge_run(P, Q, ridxc1, agg_o, HALF_NODES, s, sidx, ridx,
                     sidx_v, ridx_v, ridxc_v, pbuf, qbuf, mbuf,
                     semp, semq, acc)


_SC_EDGE_CACHE = []


def _sc_edge(*args):
    if not _SC_EDGE_CACHE:
        mesh = plsc.VectorSubcoreMesh(core_axis_name="c",
                                      subcore_axis_name="s",
                                      num_cores=2, num_subcores=NSUB)
        _SC_EDGE_CACHE.append(pl.kernel(
            _sc_edge_body,
            out_type=jax.ShapeDtypeStruct((N_NODES, HH), jnp.float32),
            mesh=mesh,
            scratch_types=[
                pltpu.VMEM((NCHUNK, CHUNK), jnp.int32),
                pltpu.VMEM((NCHUNK, CHUNK), jnp.int32),
                pltpu.VMEM((NCHUNK, CHUNK), jnp.int32),
                pltpu.VMEM((CHUNK, HH), jnp.float32),
                pltpu.VMEM((CHUNK, HH), jnp.float32),
                pltpu.VMEM((CHUNK, HH), jnp.float32),
                pltpu.SemaphoreType.DMA,
                pltpu.SemaphoreType.DMA,
                pltpu.VMEM_SHARED((ACC_ROWS, HH), jnp.float32),
            ],
        ))
    return _SC_EDGE_CACHE[0](*args)


def kernel(nodes, edge_index, x_prev, t_idx, W_enc1, b_enc1, W_enc2, b_enc2,
           W_msg, b_msg, W_upd, b_upd, W_dec1, b_dec1, W_dec2, b_dec2,
           W_a, b_a, W_b, b_b):
    f32 = jnp.float32
    W1a = W_enc1[:H]
    W1b = jnp.zeros((HH, H), f32).at[:H + 1 + 100 - H].set(W_enc1[H:])
    onehot = jax.nn.one_hot(
        t_idx * jnp.ones((N_NODES,), jnp.int32), 100, dtype=f32)
    X2 = jnp.concatenate(
        [x_prev, onehot, jnp.zeros((N_NODES, HH - 101), f32)], axis=1)
    b1 = b_enc1.reshape(1, H)
    b2 = b_enc2.reshape(1, H)

    senders = edge_index[0]
    receivers = edge_index[1]

    def tile_pad(x, pad_val):
        x = x.reshape(NSUB, EDGES_PER_TILE)
        pad = jnp.full((NSUB, PAD_EPT - EDGES_PER_TILE), pad_val, jnp.int32)
        return jnp.concatenate([x, pad], axis=1).reshape(NSUB, NCHUNK, CHUNK)

    sidx = tile_pad(senders, 0)
    ridx = tile_pad(receivers, 0)
    ridxc0 = tile_pad(jnp.where(receivers < HALF_NODES, receivers, DUMP_ROW),
                      DUMP_ROW)
    ridxc1 = tile_pad(jnp.where(receivers >= HALF_NODES,
                                receivers - HALF_NODES, DUMP_ROW), DUMP_ROW)

    Wab = jnp.zeros((H, HH), f32).at[:, 0].set(W_a[:, 0]).at[:, 1].set(W_b[:, 0])
    bab = jnp.zeros((1, HH), f32).at[0, 0].set(b_a[0]).at[0, 1].set(b_b[0])

    def msg_w(st):
        return (W_msg[st][:H], W_msg[st][H:], b_msg[st].reshape(1, H))

    def upd_w(st):
        return (W_upd[st][:H], W_upd[st][H:], b_upd[st].reshape(1, H))

    hs = jax.ShapeDtypeStruct((N_NODES, H), f32)
    hh = jax.ShapeDtypeStruct((N_NODES, HH), f32)

    Wt, Wb_, bm = msg_w(0)
    h, P0, P1, Q0, Q1 = _tc_call(
        _tc_encode_body,
        (nodes, X2, W1a, W1b, b1, W_enc2, b2, Wt, Wb_, bm),
        (hs, hh, hh, hh, hh))

    for st in range(3):
        a0 = _sc_edge(P0, Q0, sidx, ridx, ridxc0, ridxc1)
        P1s, Q1s, a0 = lax.optimization_barrier((P1, Q1, a0))
        a1 = _sc_edge(P1s, Q1s, sidx, ridx, ridxc0, ridxc1)
        agg = jnp.concatenate([a0, a1], axis=1)
        A, B, bu = upd_w(st)
        if st < 2:
            Wt, Wb_, bm = msg_w(st + 1)
            h, P0, P1, Q0, Q1 = _tc_call(
                _tc_update_body,
                (h, agg, A, B, bu, Wt, Wb_, bm),
                (hs, hh, hh, hh, hh))
        else:
            (ab,) = _tc_call(
                _tc_decode_body,
                (h, agg, A, B, bu,
                 W_dec1, b_dec1.reshape(1, H), W_dec2, b_dec2.reshape(1, H),
                 Wab, bab),
                (jax.ShapeDtypeStruct((N_NODES, HH), f32),))

    return ab[:, 0], ab[:, 1]
